# Initial kernel scaffold; baseline (speedup 1.0000x reference)
import jax
import jax.numpy as jnp
from jax.experimental import pallas as pl


def kernel(edges, labels, word_embeddings, W1, b1, W2, b2):
    raise NotImplementedError("write your pallas kernel here")


# speedup vs baseline: 3.9783x; 3.9783x over previous
import functools

import jax
import jax.numpy as jnp
from jax import lax
from jax.experimental import pallas as pl
from jax.experimental.pallas import tpu as pltpu
from jax.experimental.pallas import tpu_sc as plsc

EMBED = 30
D_PAD = 32
B_EDGES = 800000
NUM_CORES = 2
NUM_SUBCORES = 16
NW = NUM_CORES * NUM_SUBCORES
CHUNK = 1024
N_CHUNKS = 25
B_PER_W = CHUNK * N_CHUNKS
B_PAD = B_PER_W * NW
STREAM = 128
N_STREAMS = CHUNK // STREAM


def _sc_gather_mul(table, src2d, dst2d):
    mesh = plsc.VectorSubcoreMesh(core_axis_name="c", subcore_axis_name="s")

    @functools.partial(
        pl.kernel,
        mesh=mesh,
        out_type=jax.ShapeDtypeStruct((B_PAD, D_PAD), jnp.float32),
        scratch_types=[
            pltpu.VMEM((N_STREAMS, STREAM), jnp.int32),
            pltpu.VMEM((N_STREAMS, STREAM), jnp.int32),
            pltpu.VMEM((CHUNK, D_PAD), jnp.float32),
            pltpu.VMEM((CHUNK, D_PAD), jnp.float32),
            pltpu.SemaphoreType.DMA,
            pltpu.SemaphoreType.DMA,
        ],
        compiler_params=pltpu.CompilerParams(use_tc_tiling_on_sc=False),
    )
    def k(table_hbm, src_hbm, dst_hbm, out_hbm, sidx, didx, srows, drows,
          sem_g, sem_i):
        wid = lax.axis_index("s") * NUM_CORES + lax.axis_index("c")
        row0 = wid * (B_PER_W // STREAM)
        for c in range(N_CHUNKS):
            idx_row = row0 + c * N_STREAMS
            cpi1 = pltpu.async_copy(
                src_hbm.at[pl.ds(idx_row, N_STREAMS)], sidx, sem_i)
            cpi2 = pltpu.async_copy(
                dst_hbm.at[pl.ds(idx_row, N_STREAMS)], didx, sem_i)
            cpi1.wait()
            cpi2.wait()
            gathers = []
            for j in range(N_STREAMS):
                gathers.append(pltpu.async_copy(
                    table_hbm.at[sidx.at[j]],
                    srows.at[pl.ds(j * STREAM, STREAM)], sem_g))
                gathers.append(pltpu.async_copy(
                    table_hbm.at[didx.at[j]],
                    drows.at[pl.ds(j * STREAM, STREAM)], sem_g))
            for cp in gathers:
                cp.wait()

            def body(i, _):
                a0 = srows[i, pl.ds(0, 16)]
                b0 = drows[i, pl.ds(0, 16)]
                srows[i, pl.ds(0, 16)] = a0 * b0
                a1 = srows[i, pl.ds(16, 16)]
                b1 = drows[i, pl.ds(16, 16)]
                srows[i, pl.ds(16, 16)] = a1 * b1
                return 0

            lax.fori_loop(0, CHUNK, body, 0)
            base = wid * B_PER_W + c * CHUNK
            pltpu.sync_copy(srows, out_hbm.at[pl.ds(base, CHUNK)])

    return k(table, src2d, dst2d)


PACK = 4


def _tc_mlp(e4, w1blk, b1t, wdsel):
    grid = 32
    bt = B_PAD // PACK // grid

    def body(e_ref, w1_ref, b1_ref, wd_ref, out_ref):
        e = e_ref[...]
        h = jnp.dot(e, w1_ref[...], preferred_element_type=jnp.float32)
        h = jnp.maximum(h + b1_ref[...], 0.0)
        out_ref[...] = jnp.dot(h, wd_ref[...],
                               preferred_element_type=jnp.float32)

    return pl.pallas_call(
        body,
        grid=(grid,),
        in_specs=[
            pl.BlockSpec((bt, 128), lambda i: (i, 0)),
            pl.BlockSpec((128, 128), lambda i: (0, 0)),
            pl.BlockSpec((1, 128), lambda i: (0, 0)),
            pl.BlockSpec((128, PACK), lambda i: (0, 0)),
        ],
        out_specs=pl.BlockSpec((bt, PACK), lambda i: (i, 0)),
        out_shape=jax.ShapeDtypeStruct((B_PAD // PACK, PACK), jnp.float32),
    )(e4, w1blk, b1t, wdsel)


def _tc_loss(dmat, labf, bdiff):
    grid = 8
    rows = B_PAD // 128
    br = rows // grid

    def body(d_ref, l_ref, bd_ref, out_ref):
        d = d_ref[...] + bd_ref[0, 0]
        p0 = jnp.where(d >= 0.0,
                       1.0 / (1.0 + jnp.exp(-d)),
                       jnp.exp(d) / (1.0 + jnp.exp(d)))
        p1 = 1.0 - p0
        lse = jnp.log(jnp.exp(p0) + jnp.exp(p1))
        lab = l_ref[...]
        psel = p0 + lab * (1.0 - 2.0 * p0)
        loss_vec = lse - psel
        step = pl.program_id(0)
        row = (lax.broadcasted_iota(jnp.int32, (br, 128), 0) * 128
               + lax.broadcasted_iota(jnp.int32, (br, 128), 1)
               + step * br * 128)
        loss_vec = jnp.where(row < B_EDGES, loss_vec, 0.0)
        s = jnp.sum(loss_vec)

        @pl.when(step == 0)
        def _():
            out_ref[0, 0] = 0.0

        out_ref[0, 0] += s

    return pl.pallas_call(
        body,
        grid=(grid,),
        in_specs=[
            pl.BlockSpec((br, 128), lambda i: (i, 0)),
            pl.BlockSpec((br, 128), lambda i: (i, 0)),
            pl.BlockSpec(memory_space=pltpu.SMEM),
        ],
        out_specs=pl.BlockSpec(memory_space=pltpu.SMEM),
        out_shape=jax.ShapeDtypeStruct((1, 1), jnp.float32),
    )(dmat, labf, bdiff)


def kernel(edges, labels, word_embeddings, W1, b1, W2, b2):
    edges = edges.astype(jnp.int32)
    src = jnp.pad(edges[:, 0], (0, B_PAD - B_EDGES)).reshape(
        B_PAD // STREAM, STREAM)
    dst = jnp.pad(edges[:, 1], (0, B_PAD - B_EDGES)).reshape(
        B_PAD // STREAM, STREAM)
    table = jnp.pad(word_embeddings.astype(jnp.float32),
                    ((0, 0), (0, D_PAD - EMBED)))
    eemb = _sc_gather_mul(table, src, dst)

    w1p = jnp.pad(W1, ((0, D_PAD - EMBED), (0, D_PAD - EMBED)))
    b1p = jnp.pad(b1, (0, D_PAD - EMBED))
    wdp = jnp.pad(W2[:, 0] - W2[:, 1], (0, D_PAD - EMBED))
    eye4 = jnp.eye(PACK, dtype=jnp.float32)
    w1blk = jnp.kron(eye4, w1p)
    b1t = jnp.tile(b1p, PACK)[None, :]
    wdsel = jnp.kron(eye4, wdp[:, None])
    e4 = eemb.reshape(B_PAD // PACK, PACK * D_PAD)
    dcol = _tc_mlp(e4, w1blk, b1t, wdsel)
    dmat = dcol.reshape(B_PAD // 128, 128)

    labf = jnp.pad(labels.astype(jnp.float32), (0, B_PAD - B_EDGES)).reshape(
        B_PAD // 128, 128)
    bdiff = (b2[0] - b2[1]).reshape(1, 1)
    out = _tc_loss(dmat, labf, bdiff)
    return out[0, 0] / jnp.float32(B_EDGES)


# speedup vs baseline: 4.5099x; 1.1336x over previous
import functools

import jax
import jax.numpy as jnp
from jax import lax
from jax.experimental import pallas as pl
from jax.experimental.pallas import tpu as pltpu
from jax.experimental.pallas import tpu_sc as plsc

EMBED = 30
D_PAD = 32
B_EDGES = 800000
NUM_CORES = 2
NUM_SUBCORES = 16
NW = NUM_CORES * NUM_SUBCORES
CHUNK = 512
N_CHUNKS = 50
B_PER_W = CHUNK * N_CHUNKS
B_PAD = B_PER_W * NW
STREAM = 128
N_STREAMS = CHUNK // STREAM
IDX_ROWS = B_PER_W // STREAM


def _sc_gather_mul(table, src2d, dst2d):
    mesh = plsc.VectorSubcoreMesh(core_axis_name="c", subcore_axis_name="s")

    @functools.partial(
        pl.kernel,
        mesh=mesh,
        out_type=jax.ShapeDtypeStruct((B_PAD, D_PAD), jnp.float32),
        scratch_types=[
            pltpu.VMEM((IDX_ROWS, STREAM), jnp.int32),
            pltpu.VMEM((IDX_ROWS, STREAM), jnp.int32),
            pltpu.VMEM((CHUNK, D_PAD), jnp.float32),
            pltpu.VMEM((CHUNK, D_PAD), jnp.float32),
            pltpu.VMEM((CHUNK, D_PAD), jnp.float32),
            pltpu.VMEM((CHUNK, D_PAD), jnp.float32),
            pltpu.SemaphoreType.DMA,
            pltpu.SemaphoreType.DMA,
        ],
        compiler_params=pltpu.CompilerParams(use_tc_tiling_on_sc=False),
    )
    def k(table_hbm, src_hbm, dst_hbm, out_hbm, sidx, didx,
          srows0, drows0, srows1, drows1, sem0, sem1):
        wid = lax.axis_index("s") * NUM_CORES + lax.axis_index("c")
        pltpu.sync_copy(src_hbm.at[pl.ds(wid * IDX_ROWS, IDX_ROWS)], sidx)
        pltpu.sync_copy(dst_hbm.at[pl.ds(wid * IDX_ROWS, IDX_ROWS)], didx)
        bufs = ((srows0, drows0, sem0), (srows1, drows1, sem1))

        def fire(c):
            srows, drows, sem = bufs[c % 2]
            cps = []
            for j in range(N_STREAMS):
                row = c * N_STREAMS + j
                cps.append(pltpu.async_copy(
                    table_hbm.at[sidx.at[row]],
                    srows.at[pl.ds(j * STREAM, STREAM)], sem))
                cps.append(pltpu.async_copy(
                    table_hbm.at[didx.at[row]],
                    drows.at[pl.ds(j * STREAM, STREAM)], sem))
            return cps

        pend = {0: fire(0), 1: fire(1)}
        for c in range(N_CHUNKS):
            srows, drows, _ = bufs[c % 2]
            for cp in pend.pop(c):
                cp.wait()

            @plsc.parallel_loop(0, CHUNK, 1, unroll=4)
            def _(i):
                a0 = srows[i, pl.ds(0, 16)]
                b0 = drows[i, pl.ds(0, 16)]
                srows[i, pl.ds(0, 16)] = a0 * b0
                a1 = srows[i, pl.ds(16, 16)]
                b1 = drows[i, pl.ds(16, 16)]
                srows[i, pl.ds(16, 16)] = a1 * b1

            base = wid * B_PER_W + c * CHUNK
            pltpu.sync_copy(srows, out_hbm.at[pl.ds(base, CHUNK)])
            if c + 2 < N_CHUNKS:
                pend[c + 2] = fire(c + 2)

    return k(table, src2d, dst2d)


PACK = 4


def _tc_mlp(e4, w1blk, b1t, wdsel):
    grid = 32
    bt = B_PAD // PACK // grid

    def body(e_ref, w1_ref, b1_ref, wd_ref, out_ref):
        e = e_ref[...]
        h = jnp.dot(e, w1_ref[...], preferred_element_type=jnp.float32)
        h = jnp.maximum(h + b1_ref[...], 0.0)
        out_ref[...] = jnp.dot(h, wd_ref[...],
                               preferred_element_type=jnp.float32)

    return pl.pallas_call(
        body,
        grid=(grid,),
        in_specs=[
            pl.BlockSpec((bt, 128), lambda i: (i, 0)),
            pl.BlockSpec((128, 128), lambda i: (0, 0)),
            pl.BlockSpec((1, 128), lambda i: (0, 0)),
            pl.BlockSpec((128, PACK), lambda i: (0, 0)),
        ],
        out_specs=pl.BlockSpec((bt, PACK), lambda i: (i, 0)),
        out_shape=jax.ShapeDtypeStruct((B_PAD // PACK, PACK), jnp.float32),
    )(e4, w1blk, b1t, wdsel)


def _tc_loss(dmat, labf, bdiff):
    grid = 8
    rows = B_PAD // 128
    br = rows // grid

    def body(d_ref, l_ref, bd_ref, out_ref):
        d = d_ref[...] + bd_ref[0, 0]
        p0 = jnp.where(d >= 0.0,
                       1.0 / (1.0 + jnp.exp(-d)),
                       jnp.exp(d) / (1.0 + jnp.exp(d)))
        p1 = 1.0 - p0
        lse = jnp.log(jnp.exp(p0) + jnp.exp(p1))
        lab = l_ref[...]
        psel = p0 + lab * (1.0 - 2.0 * p0)
        loss_vec = lse - psel
        step = pl.program_id(0)
        row = (lax.broadcasted_iota(jnp.int32, (br, 128), 0) * 128
               + lax.broadcasted_iota(jnp.int32, (br, 128), 1)
               + step * br * 128)
        loss_vec = jnp.where(row < B_EDGES, loss_vec, 0.0)
        s = jnp.sum(loss_vec)

        @pl.when(step == 0)
        def _():
            out_ref[0, 0] = 0.0

        out_ref[0, 0] += s

    return pl.pallas_call(
        body,
        grid=(grid,),
        in_specs=[
            pl.BlockSpec((br, 128), lambda i: (i, 0)),
            pl.BlockSpec((br, 128), lambda i: (i, 0)),
            pl.BlockSpec(memory_space=pltpu.SMEM),
        ],
        out_specs=pl.BlockSpec(memory_space=pltpu.SMEM),
        out_shape=jax.ShapeDtypeStruct((1, 1), jnp.float32),
    )(dmat, labf, bdiff)


def kernel(edges, labels, word_embeddings, W1, b1, W2, b2):
    edges = edges.astype(jnp.int32)
    src = jnp.pad(edges[:, 0], (0, B_PAD - B_EDGES)).reshape(
        B_PAD // STREAM, STREAM)
    dst = jnp.pad(edges[:, 1], (0, B_PAD - B_EDGES)).reshape(
        B_PAD // STREAM, STREAM)
    table = jnp.pad(word_embeddings.astype(jnp.float32),
                    ((0, 0), (0, D_PAD - EMBED)))
    eemb = _sc_gather_mul(table, src, dst)

    w1p = jnp.pad(W1, ((0, D_PAD - EMBED), (0, D_PAD - EMBED)))
    b1p = jnp.pad(b1, (0, D_PAD - EMBED))
    wdp = jnp.pad(W2[:, 0] - W2[:, 1], (0, D_PAD - EMBED))
    eye4 = jnp.eye(PACK, dtype=jnp.float32)
    w1blk = jnp.kron(eye4, w1p)
    b1t = jnp.tile(b1p, PACK)[None, :]
    wdsel = jnp.kron(eye4, wdp[:, None])
    e4 = eemb.reshape(B_PAD // PACK, PACK * D_PAD)
    dcol = _tc_mlp(e4, w1blk, b1t, wdsel)
    dmat = dcol.reshape(B_PAD // 128, 128)

    labf = jnp.pad(labels.astype(jnp.float32), (0, B_PAD - B_EDGES)).reshape(
        B_PAD // 128, 128)
    bdiff = (b2[0] - b2[1]).reshape(1, 1)
    out = _tc_loss(dmat, labf, bdiff)
    return out[0, 0] / jnp.float32(B_EDGES)


# speedup vs baseline: 4.5217x; 1.0026x over previous
import functools

import jax
import jax.numpy as jnp
from jax import lax
from jax.experimental import pallas as pl
from jax.experimental.pallas import tpu as pltpu
from jax.experimental.pallas import tpu_sc as plsc

EMBED = 30
D_PAD = 32
B_EDGES = 800000
NUM_CORES = 2
NUM_SUBCORES = 16
NW = NUM_CORES * NUM_SUBCORES
CHUNK = 512
N_CHUNKS = 50
B_PER_W = CHUNK * N_CHUNKS
B_PAD = B_PER_W * NW
STREAM = 128
N_STREAMS = CHUNK // STREAM
IDX_ROWS = B_PER_W // STREAM


def _sc_gather_mul(table, src2d, dst2d):
    mesh = plsc.VectorSubcoreMesh(core_axis_name="c", subcore_axis_name="s")

    @functools.partial(
        pl.kernel,
        mesh=mesh,
        out_type=jax.ShapeDtypeStruct((B_PAD, D_PAD), jnp.float32),
        scratch_types=[
            pltpu.VMEM((B_PER_W,), jnp.int32),
            pltpu.VMEM((B_PER_W,), jnp.int32),
            pltpu.VMEM((CHUNK, D_PAD), jnp.float32),
            pltpu.VMEM((CHUNK, D_PAD), jnp.float32),
            pltpu.VMEM((CHUNK, D_PAD), jnp.float32),
            pltpu.VMEM((CHUNK, D_PAD), jnp.float32),
            pltpu.SemaphoreType.DMA,
            pltpu.SemaphoreType.DMA,
        ],
        compiler_params=pltpu.CompilerParams(use_tc_tiling_on_sc=False),
    )
    def k(table_hbm, src_hbm, dst_hbm, out_hbm, sidx, didx,
          srows0, drows0, srows1, drows1, sem0, sem1):
        wid = lax.axis_index("s") * NUM_CORES + lax.axis_index("c")
        pltpu.sync_copy(src_hbm.at[pl.ds(wid * B_PER_W, B_PER_W)], sidx)
        pltpu.sync_copy(dst_hbm.at[pl.ds(wid * B_PER_W, B_PER_W)], didx)
        bufs = ((srows0, drows0, sem0), (srows1, drows1, sem1))

        def fire(c):
            srows, drows, sem = bufs[c % 2]
            return [
                pltpu.async_copy(
                    table_hbm.at[sidx.at[pl.ds(c * CHUNK, CHUNK)]],
                    srows, sem),
                pltpu.async_copy(
                    table_hbm.at[didx.at[pl.ds(c * CHUNK, CHUNK)]],
                    drows, sem),
            ]

        pend = {0: fire(0), 1: fire(1)}
        for c in range(N_CHUNKS):
            srows, drows, _ = bufs[c % 2]
            for cp in pend.pop(c):
                cp.wait()

            @plsc.parallel_loop(0, CHUNK, 1, unroll=4)
            def _(i):
                a0 = srows[i, pl.ds(0, 16)]
                b0 = drows[i, pl.ds(0, 16)]
                srows[i, pl.ds(0, 16)] = a0 * b0
                a1 = srows[i, pl.ds(16, 16)]
                b1 = drows[i, pl.ds(16, 16)]
                srows[i, pl.ds(16, 16)] = a1 * b1

            base = wid * B_PER_W + c * CHUNK
            pltpu.sync_copy(srows, out_hbm.at[pl.ds(base, CHUNK)])
            if c + 2 < N_CHUNKS:
                pend[c + 2] = fire(c + 2)

    return k(table, src2d, dst2d)


PACK = 4


def _tc_mlp(e4, w1blk, b1t, wdsel):
    grid = 32
    bt = B_PAD // PACK // grid

    def body(e_ref, w1_ref, b1_ref, wd_ref, out_ref):
        e = e_ref[...]
        h = jnp.dot(e, w1_ref[...], preferred_element_type=jnp.float32)
        h = jnp.maximum(h + b1_ref[...], 0.0)
        out_ref[...] = jnp.dot(h, wd_ref[...],
                               preferred_element_type=jnp.float32)

    return pl.pallas_call(
        body,
        grid=(grid,),
        in_specs=[
            pl.BlockSpec((bt, 128), lambda i: (i, 0)),
            pl.BlockSpec((128, 128), lambda i: (0, 0)),
            pl.BlockSpec((1, 128), lambda i: (0, 0)),
            pl.BlockSpec((128, PACK), lambda i: (0, 0)),
        ],
        out_specs=pl.BlockSpec((bt, PACK), lambda i: (i, 0)),
        out_shape=jax.ShapeDtypeStruct((B_PAD // PACK, PACK), jnp.float32),
    )(e4, w1blk, b1t, wdsel)


def _tc_loss(dmat, labf, bdiff):
    grid = 8
    rows = B_PAD // 128
    br = rows // grid

    def body(d_ref, l_ref, bd_ref, out_ref):
        d = d_ref[...] + bd_ref[0, 0]
        p0 = jnp.where(d >= 0.0,
                       1.0 / (1.0 + jnp.exp(-d)),
                       jnp.exp(d) / (1.0 + jnp.exp(d)))
        p1 = 1.0 - p0
        lse = jnp.log(jnp.exp(p0) + jnp.exp(p1))
        lab = l_ref[...]
        psel = p0 + lab * (1.0 - 2.0 * p0)
        loss_vec = lse - psel
        step = pl.program_id(0)
        row = (lax.broadcasted_iota(jnp.int32, (br, 128), 0) * 128
               + lax.broadcasted_iota(jnp.int32, (br, 128), 1)
               + step * br * 128)
        loss_vec = jnp.where(row < B_EDGES, loss_vec, 0.0)
        s = jnp.sum(loss_vec)

        @pl.when(step == 0)
        def _():
            out_ref[0, 0] = 0.0

        out_ref[0, 0] += s

    return pl.pallas_call(
        body,
        grid=(grid,),
        in_specs=[
            pl.BlockSpec((br, 128), lambda i: (i, 0)),
            pl.BlockSpec((br, 128), lambda i: (i, 0)),
            pl.BlockSpec(memory_space=pltpu.SMEM),
        ],
        out_specs=pl.BlockSpec(memory_space=pltpu.SMEM),
        out_shape=jax.ShapeDtypeStruct((1, 1), jnp.float32),
    )(dmat, labf, bdiff)


def kernel(edges, labels, word_embeddings, W1, b1, W2, b2):
    edges = edges.astype(jnp.int32)
    src = jnp.pad(edges[:, 0], (0, B_PAD - B_EDGES))
    dst = jnp.pad(edges[:, 1], (0, B_PAD - B_EDGES))
    table = jnp.pad(word_embeddings.astype(jnp.float32),
                    ((0, 0), (0, D_PAD - EMBED)))
    eemb = _sc_gather_mul(table, src, dst)

    w1p = jnp.pad(W1, ((0, D_PAD - EMBED), (0, D_PAD - EMBED)))
    b1p = jnp.pad(b1, (0, D_PAD - EMBED))
    wdp = jnp.pad(W2[:, 0] - W2[:, 1], (0, D_PAD - EMBED))
    eye4 = jnp.eye(PACK, dtype=jnp.float32)
    w1blk = jnp.kron(eye4, w1p)
    b1t = jnp.tile(b1p, PACK)[None, :]
    wdsel = jnp.kron(eye4, wdp[:, None])
    e4 = eemb.reshape(B_PAD // PACK, PACK * D_PAD)
    dcol = _tc_mlp(e4, w1blk, b1t, wdsel)
    dmat = dcol.reshape(B_PAD // 128, 128)

    labf = jnp.pad(labels.astype(jnp.float32), (0, B_PAD - B_EDGES)).reshape(
        B_PAD // 128, 128)
    bdiff = (b2[0] - b2[1]).reshape(1, 1)
    out = _tc_loss(dmat, labf, bdiff)
    return out[0, 0] / jnp.float32(B_EDGES)
